# baseline (speedup 1.0000x reference)
import functools

import jax
import jax.numpy as jnp
from jax import lax
from jax.experimental import pallas as pl
from jax.experimental.pallas import tpu as pltpu
from jax.experimental.pallas import tpu_sc as plsc

N = 10000
H = 128
LANES = 64
NW = 32
N_PAD = 10240
RPT = N_PAD // 16
DEAD = N_PAD - 8
BLK = 512


def _k1_body(x_ref, w_ref, b_ref, yg_ref, z_ref):
    xw = jnp.dot(x_ref[...], w_ref[...], preferred_element_type=jnp.float32)
    yg_ref[...] = xw[:, :H]
    z_ref[...] = xw[:, H:] + b_ref[...]


def _k2_body(agg_ref, dp_ref, z1_ref, g_ref, be_ref, w_ref, b2_ref,
             yg2_ref, z2_ref, rdeg_ref):
    p = agg_ref[...]
    dp = dp_ref[...]
    ssum = p[0] + p[1]
    deg = jnp.sum(dp, axis=0)[:, None]
    rdeg = 1.0 / jnp.maximum(deg, 1.0)
    rdeg_ref[...] = jnp.broadcast_to(rdeg, (rdeg.shape[0], 16))
    pre = ssum * rdeg + z1_ref[...]
    mu = jnp.mean(pre, axis=-1, keepdims=True)
    d = pre - mu
    var = jnp.mean(d * d, axis=-1, keepdims=True)
    h = d * lax.rsqrt(var + 1e-5) * g_ref[...] + be_ref[...]
    h = jnp.maximum(h, 0.0)
    hw = jnp.dot(h, w_ref[...], preferred_element_type=jnp.float32)
    yg2_ref[...] = hw[:, :H]
    z2_ref[...] = hw[:, H:] + b2_ref[...]


def _k3_body(agg2_ref, rdeg_ref, z2_ref, out_ref):
    p = agg2_ref[...]
    rdeg = rdeg_ref[...][:, 0:1]
    out_ref[...] = (p[0] + p[1]) * rdeg + z2_ref[...]


def _sc_mesh():
    return plsc.VectorSubcoreMesh(
        core_axis_name="c", subcore_axis_name="s", num_cores=2,
        num_subcores=16)


def _make_sc_agg(n_chunks):
    @functools.partial(
        pl.kernel,
        out_type=jax.ShapeDtypeStruct((2, N_PAD, H), jnp.float32),
        mesh=_sc_mesh(),
        scratch_types=[
            pltpu.VMEM((n_chunks // 2 * LANES,), jnp.int32),
            pltpu.VMEM((n_chunks // 2, LANES), jnp.int32),
            pltpu.VMEM((4, LANES, H), jnp.float32),
            pltpu.VMEM_SHARED((N_PAD, H), jnp.float32),
            pltpu.SemaphoreType.DMA,
            pltpu.SemaphoreType.DMA,
        ],
    )
    def sc_agg(y_hbm, src_hbm, dst_hbm, zeros_hbm, out_hbm,
               src_v, dst_v, rows_v, acc_sh, sem_g, sem_s):
        c = lax.axis_index("c")
        s = lax.axis_index("s")
        wid = s * 2 + c
        pltpu.sync_copy(zeros_hbm, acc_sh.at[pl.ds(s * RPT, RPT)])
        plsc.subcore_barrier()
        bufs = [rows_v.at[b] for b in range(4)]

        def gather(j, b):
            pltpu.async_copy(
                y_hbm.at[src_v.at[pl.ds(j * LANES, LANES)]], bufs[b], sem_g)

        def wait_g(j, b):
            pltpu.make_async_copy(
                y_hbm.at[src_v.at[pl.ds(j * LANES, LANES)]], bufs[b],
                sem_g).wait()

        def scatter(j, b):
            pltpu.async_copy(bufs[b], acc_sh.at[dst_v.at[j]], sem_s, add=True)

        def wait_s(j, b):
            pltpu.make_async_copy(bufs[b], acc_sh.at[dst_v.at[j]], sem_s).wait()

        half = n_chunks // 2
        for p in range(2):
            pltpu.sync_copy(
                src_hbm.at[wid, pl.ds(p * half * LANES, half * LANES)], src_v)
            pltpu.sync_copy(dst_hbm.at[wid, pl.ds(p * half, half)], dst_v)
            gather(0, 0)
            gather(1, 1)
            wait_g(0, 0)
            gather(2, 2)
            scatter(0, 0)
            wait_g(1, 1)
            gather(3, 3)
            scatter(1, 1)
            wait_g(2, 2)
            wait_s(0, 0)
            gather(4, 0)
            scatter(2, 2)
            wait_g(3, 3)
            wait_s(1, 1)
            gather(5, 1)
            scatter(3, 3)

            def quad(k, carry):
                j0 = 4 * k
                wait_g(j0, 0)
                wait_s(j0 - 2, 2)
                gather(j0 + 2, 2)
                scatter(j0, 0)
                wait_g(j0 + 1, 1)
                wait_s(j0 - 1, 3)
                gather(j0 + 3, 3)
                scatter(j0 + 1, 1)
                wait_g(j0 + 2, 2)
                wait_s(j0, 0)
                gather(j0 + 4, 0)
                scatter(j0 + 2, 2)
                wait_g(j0 + 3, 3)
                wait_s(j0 + 1, 1)
                gather(j0 + 5, 1)
                scatter(j0 + 3, 3)
                return carry

            lax.fori_loop(1, half // 4 - 1, quad, 0)
            j0 = half - 4
            wait_g(j0, 0)
            wait_s(j0 - 2, 2)
            gather(j0 + 2, 2)
            scatter(j0, 0)
            wait_g(j0 + 1, 1)
            wait_s(j0 - 1, 3)
            gather(j0 + 3, 3)
            scatter(j0 + 1, 1)
            wait_g(j0 + 2, 2)
            wait_s(j0, 0)
            scatter(j0 + 2, 2)
            wait_g(j0 + 3, 3)
            wait_s(j0 + 1, 1)
            scatter(j0 + 3, 3)
            wait_s(j0 + 2, 2)
            wait_s(j0 + 3, 3)
        plsc.subcore_barrier()
        pltpu.sync_copy(acc_sh.at[pl.ds(s * RPT, RPT)],
                        out_hbm.at[c, pl.ds(s * RPT, RPT)])

    return sc_agg


_HALF_N = N_PAD // 2


def _make_sc_deg(n_chunks):
    @functools.partial(
        pl.kernel,
        out_type=jax.ShapeDtypeStruct((NW, N_PAD), jnp.float32),
        mesh=_sc_mesh(),
        compiler_params=pltpu.CompilerParams(needs_layout_passes=False),
        scratch_types=[
            pltpu.VMEM((n_chunks, LANES), jnp.int32),
            pltpu.VMEM((16 * _HALF_N + 16,), jnp.float32),
            pltpu.VMEM((N_PAD,), jnp.float32),
        ],
    )
    def sc_deg(dst_hbm, zeros_hbm, out_hbm, dst_v, hist_v, deg_v):
        c = lax.axis_index("c")
        s = lax.axis_index("s")
        wid = s * 2 + c
        pltpu.sync_copy(dst_hbm.at[wid], dst_v)
        lane = lax.iota(jnp.int32, 16)
        lane_off = lane * _HALF_N
        trash = 16 * _HALF_N + lane
        ones = jnp.ones((16,), jnp.float32)

        for half in range(2):
            pltpu.sync_copy(zeros_hbm, hist_v.at[pl.ds(0, 16 * _HALF_N)])
            base = half * _HALF_N

            def count(j, carry):
                for k in range(LANES // 16):
                    d = dst_v[j, pl.ds(k * 16, 16)] - base
                    m = (d >= 0) & (d < _HALF_N)
                    idx = jnp.where(m, lane_off + d, trash)
                    cur = plsc.load_gather(hist_v, [idx])
                    plsc.store_scatter(hist_v, [idx], cur + ones)
                return carry

            lax.fori_loop(0, n_chunks, count, 0)

            def lanesum(g, carry):
                acc = hist_v[pl.ds(g * 16, 16)]
                for l in range(1, 16):
                    acc = acc + hist_v[pl.ds(l * _HALF_N + g * 16, 16)]
                deg_v[pl.ds(base + g * 16, 16)] = acc
                return carry

            lax.fori_loop(0, _HALF_N // 16, lanesum, 0)

        pltpu.sync_copy(deg_v, out_hbm.at[wid])

    return sc_deg


def _tc_call(body, in_arrays, in_specs, out_shapes, out_specs, grid):
    return pl.pallas_call(
        body, grid=grid, in_specs=in_specs,
        out_specs=out_specs, out_shape=out_shapes,
    )(*in_arrays)


def kernel(x, edge_index, W1_l, b1_l, W1_r, gamma, beta, W2_l, b2_l, W2_r):
    n, in_dim = x.shape
    e = edge_index.shape[1]
    n_chunks = -(-e // (NW * LANES))
    n_chunks = -(-n_chunks // 8) * 8
    e_pad = NW * n_chunks * LANES

    x = jnp.pad(x, ((0, N_PAD - n), (0, 0)))
    ei = edge_index.astype(jnp.int32)
    pad = e_pad - e
    pad_src = jnp.arange(pad, dtype=jnp.int32) % N
    src2d = jnp.concatenate(
        [ei[0], pad_src]).reshape(NW, n_chunks * LANES)
    pad_dst = N + jnp.arange(pad, dtype=jnp.int32) % (N_PAD - N)
    dst3d = jnp.concatenate(
        [ei[1], pad_dst]).reshape(NW, n_chunks, LANES)
    zeros_rp = jnp.zeros((RPT, H), jnp.float32)
    zeros_hist = jnp.zeros((16 * _HALF_N,), jnp.float32)
    wt1 = jnp.concatenate([W1_l, W1_r], axis=0).T
    wt2 = jnp.concatenate([W2_l, W2_r], axis=0).T
    b1r = b1_l.reshape(1, H)
    b2r = b2_l.reshape(1, H)
    gr = gamma.reshape(1, H)
    br = beta.reshape(1, H)

    grid = (N_PAD // BLK,)
    row_spec = lambda w: pl.BlockSpec((BLK, w), lambda i: (i, 0))
    full_spec = lambda a: pl.BlockSpec(a.shape, lambda i: (0, 0))
    part_spec = pl.BlockSpec((2, BLK, H), lambda i: (0, i, 0))

    yg1, z1 = _tc_call(
        _k1_body, (x, wt1, b1r),
        [row_spec(in_dim), full_spec(wt1), full_spec(b1r)],
        [jax.ShapeDtypeStruct((N_PAD, H), jnp.float32),
         jax.ShapeDtypeStruct((N_PAD, H), jnp.float32)],
        [row_spec(H), row_spec(H)], grid)

    degp = _make_sc_deg(n_chunks)(dst3d, zeros_hist)
    agg1 = _make_sc_agg(n_chunks)(yg1, src2d, dst3d, zeros_rp)

    deg_spec = pl.BlockSpec((NW, BLK), lambda i: (0, i))
    yg2, z2, rdeg = _tc_call(
        _k2_body, (agg1, degp, z1, gr, br, wt2, b2r),
        [part_spec, deg_spec, row_spec(H), full_spec(gr), full_spec(br),
         full_spec(wt2), full_spec(b2r)],
        [jax.ShapeDtypeStruct((N_PAD, H), jnp.float32),
         jax.ShapeDtypeStruct((N_PAD, H), jnp.float32),
         jax.ShapeDtypeStruct((N_PAD, 16), jnp.float32)],
        [row_spec(H), row_spec(H), row_spec(16)], grid)

    agg2 = _make_sc_agg(n_chunks)(yg2, src2d, dst3d, zeros_rp)

    out = _tc_call(
        _k3_body, (agg2, rdeg, z2),
        [part_spec, row_spec(16), row_spec(H)],
        jax.ShapeDtypeStruct((N_PAD, H), jnp.float32),
        row_spec(H), grid)
    return out[:n]


# speedup vs baseline: 1.0297x; 1.0297x over previous
import functools

import jax
import jax.numpy as jnp
from jax import lax
from jax.experimental import pallas as pl
from jax.experimental.pallas import tpu as pltpu
from jax.experimental.pallas import tpu_sc as plsc

N = 10000
H = 128
LANES = 64
NW = 32
N_PAD = 10240
RPT = N_PAD // 16
DEAD = N_PAD - 8
BLK = 1024


def _k1_body(x_ref, w_ref, b_ref, yg_ref, z_ref):
    xw = jnp.dot(x_ref[...], w_ref[...], preferred_element_type=jnp.float32)
    yg_ref[...] = xw[:, :H]
    z_ref[...] = xw[:, H:] + b_ref[...]


def _k2_body(agg_ref, dp_ref, z1_ref, g_ref, be_ref, w_ref, b2_ref,
             yg2_ref, z2_ref, rdeg_ref):
    p = agg_ref[...]
    dp = dp_ref[...]
    ssum = p[0] + p[1]
    deg = jnp.sum(dp, axis=0)[:, None]
    rdeg = 1.0 / jnp.maximum(deg, 1.0)
    rdeg_ref[...] = jnp.broadcast_to(rdeg, (rdeg.shape[0], 16))
    pre = ssum * rdeg + z1_ref[...]
    mu = jnp.mean(pre, axis=-1, keepdims=True)
    d = pre - mu
    var = jnp.mean(d * d, axis=-1, keepdims=True)
    h = d * lax.rsqrt(var + 1e-5) * g_ref[...] + be_ref[...]
    h = jnp.maximum(h, 0.0)
    hw = jnp.dot(h, w_ref[...], preferred_element_type=jnp.float32)
    yg2_ref[...] = hw[:, :H]
    z2_ref[...] = hw[:, H:] + b2_ref[...]


def _k3_body(agg2_ref, rdeg_ref, z2_ref, out_ref):
    p = agg2_ref[...]
    rdeg = rdeg_ref[...][:, 0:1]
    out_ref[...] = (p[0] + p[1]) * rdeg + z2_ref[...]


def _sc_mesh():
    return plsc.VectorSubcoreMesh(
        core_axis_name="c", subcore_axis_name="s", num_cores=2,
        num_subcores=16)


def _make_sc_agg(n_chunks):
    @functools.partial(
        pl.kernel,
        out_type=jax.ShapeDtypeStruct((2, N_PAD, H), jnp.float32),
        mesh=_sc_mesh(),
        scratch_types=[
            pltpu.VMEM((n_chunks // 2 * LANES,), jnp.int32),
            pltpu.VMEM((n_chunks // 2, LANES), jnp.int32),
            pltpu.VMEM((4, LANES, H), jnp.float32),
            pltpu.VMEM_SHARED((N_PAD, H), jnp.float32),
            pltpu.SemaphoreType.DMA,
            pltpu.SemaphoreType.DMA,
        ],
    )
    def sc_agg(y_hbm, src_hbm, dst_hbm, zeros_hbm, out_hbm,
               src_v, dst_v, rows_v, acc_sh, sem_g, sem_s):
        c = lax.axis_index("c")
        s = lax.axis_index("s")
        wid = s * 2 + c
        pltpu.sync_copy(zeros_hbm, acc_sh.at[pl.ds(s * RPT, RPT)])
        plsc.subcore_barrier()
        bufs = [rows_v.at[b] for b in range(4)]

        def gather(j, b):
            pltpu.async_copy(
                y_hbm.at[src_v.at[pl.ds(j * LANES, LANES)]], bufs[b], sem_g)

        def wait_g(j, b):
            pltpu.make_async_copy(
                y_hbm.at[src_v.at[pl.ds(j * LANES, LANES)]], bufs[b],
                sem_g).wait()

        def scatter(j, b):
            pltpu.async_copy(bufs[b], acc_sh.at[dst_v.at[j]], sem_s, add=True)

        def wait_s(j, b):
            pltpu.make_async_copy(bufs[b], acc_sh.at[dst_v.at[j]], sem_s).wait()

        half = n_chunks // 2
        for p in range(2):
            pltpu.sync_copy(
                src_hbm.at[wid, pl.ds(p * half * LANES, half * LANES)], src_v)
            pltpu.sync_copy(dst_hbm.at[wid, pl.ds(p * half, half)], dst_v)
            gather(0, 0)
            gather(1, 1)
            wait_g(0, 0)
            gather(2, 2)
            scatter(0, 0)
            wait_g(1, 1)
            gather(3, 3)
            scatter(1, 1)
            wait_g(2, 2)
            wait_s(0, 0)
            gather(4, 0)
            scatter(2, 2)
            wait_g(3, 3)
            wait_s(1, 1)
            gather(5, 1)
            scatter(3, 3)

            def quad(k, carry):
                j0 = 4 * k
                wait_g(j0, 0)
                wait_s(j0 - 2, 2)
                gather(j0 + 2, 2)
                scatter(j0, 0)
                wait_g(j0 + 1, 1)
                wait_s(j0 - 1, 3)
                gather(j0 + 3, 3)
                scatter(j0 + 1, 1)
                wait_g(j0 + 2, 2)
                wait_s(j0, 0)
                gather(j0 + 4, 0)
                scatter(j0 + 2, 2)
                wait_g(j0 + 3, 3)
                wait_s(j0 + 1, 1)
                gather(j0 + 5, 1)
                scatter(j0 + 3, 3)
                return carry

            lax.fori_loop(1, half // 4 - 1, quad, 0)
            j0 = half - 4
            wait_g(j0, 0)
            wait_s(j0 - 2, 2)
            gather(j0 + 2, 2)
            scatter(j0, 0)
            wait_g(j0 + 1, 1)
            wait_s(j0 - 1, 3)
            gather(j0 + 3, 3)
            scatter(j0 + 1, 1)
            wait_g(j0 + 2, 2)
            wait_s(j0, 0)
            scatter(j0 + 2, 2)
            wait_g(j0 + 3, 3)
            wait_s(j0 + 1, 1)
            scatter(j0 + 3, 3)
            wait_s(j0 + 2, 2)
            wait_s(j0 + 3, 3)
        plsc.subcore_barrier()
        pltpu.sync_copy(acc_sh.at[pl.ds(s * RPT, RPT)],
                        out_hbm.at[c, pl.ds(s * RPT, RPT)])

    return sc_agg


_HALF_N = N_PAD // 2


def _make_sc_deg(n_chunks):
    @functools.partial(
        pl.kernel,
        out_type=jax.ShapeDtypeStruct((NW, N_PAD), jnp.float32),
        mesh=_sc_mesh(),
        compiler_params=pltpu.CompilerParams(needs_layout_passes=False),
        scratch_types=[
            pltpu.VMEM((n_chunks, LANES), jnp.int32),
            pltpu.VMEM((16 * _HALF_N + 16,), jnp.float32),
            pltpu.VMEM((N_PAD,), jnp.float32),
        ],
    )
    def sc_deg(dst_hbm, zeros_hbm, out_hbm, dst_v, hist_v, deg_v):
        c = lax.axis_index("c")
        s = lax.axis_index("s")
        wid = s * 2 + c
        pltpu.sync_copy(dst_hbm.at[wid], dst_v)
        lane = lax.iota(jnp.int32, 16)
        lane_off = lane * _HALF_N
        trash = 16 * _HALF_N + lane
        ones = jnp.ones((16,), jnp.float32)

        for half in range(2):
            pltpu.sync_copy(zeros_hbm, hist_v.at[pl.ds(0, 16 * _HALF_N)])
            base = half * _HALF_N

            def count(j, carry):
                for k in range(LANES // 16):
                    d = dst_v[j, pl.ds(k * 16, 16)] - base
                    m = (d >= 0) & (d < _HALF_N)
                    idx = jnp.where(m, lane_off + d, trash)
                    cur = plsc.load_gather(hist_v, [idx])
                    plsc.store_scatter(hist_v, [idx], cur + ones)
                return carry

            lax.fori_loop(0, n_chunks, count, 0)

            def lanesum(g, carry):
                acc = hist_v[pl.ds(g * 16, 16)]
                for l in range(1, 16):
                    acc = acc + hist_v[pl.ds(l * _HALF_N + g * 16, 16)]
                deg_v[pl.ds(base + g * 16, 16)] = acc
                return carry

            lax.fori_loop(0, _HALF_N // 16, lanesum, 0)

        pltpu.sync_copy(deg_v, out_hbm.at[wid])

    return sc_deg


def _tc_call(body, in_arrays, in_specs, out_shapes, out_specs, grid):
    return pl.pallas_call(
        body, grid=grid, in_specs=in_specs,
        out_specs=out_specs, out_shape=out_shapes,
    )(*in_arrays)


def kernel(x, edge_index, W1_l, b1_l, W1_r, gamma, beta, W2_l, b2_l, W2_r):
    n, in_dim = x.shape
    e = edge_index.shape[1]
    n_chunks = -(-e // (NW * LANES))
    n_chunks = -(-n_chunks // 8) * 8
    e_pad = NW * n_chunks * LANES

    x = jnp.pad(x, ((0, N_PAD - n), (0, 0)))
    ei = edge_index.astype(jnp.int32)
    pad = e_pad - e
    pad_src = jnp.arange(pad, dtype=jnp.int32) % N
    src2d = jnp.concatenate(
        [ei[0], pad_src]).reshape(NW, n_chunks * LANES)
    pad_dst = N + jnp.arange(pad, dtype=jnp.int32) % (N_PAD - N)
    dst3d = jnp.concatenate(
        [ei[1], pad_dst]).reshape(NW, n_chunks, LANES)
    zeros_rp = jnp.zeros((RPT, H), jnp.float32)
    zeros_hist = jnp.zeros((16 * _HALF_N,), jnp.float32)
    wt1 = jnp.concatenate([W1_l, W1_r], axis=0).T
    wt2 = jnp.concatenate([W2_l, W2_r], axis=0).T
    b1r = b1_l.reshape(1, H)
    b2r = b2_l.reshape(1, H)
    gr = gamma.reshape(1, H)
    br = beta.reshape(1, H)

    grid = (N_PAD // BLK,)
    row_spec = lambda w: pl.BlockSpec((BLK, w), lambda i: (i, 0))
    full_spec = lambda a: pl.BlockSpec(a.shape, lambda i: (0, 0))
    part_spec = pl.BlockSpec((2, BLK, H), lambda i: (0, i, 0))

    yg1, z1 = _tc_call(
        _k1_body, (x, wt1, b1r),
        [row_spec(in_dim), full_spec(wt1), full_spec(b1r)],
        [jax.ShapeDtypeStruct((N_PAD, H), jnp.float32),
         jax.ShapeDtypeStruct((N_PAD, H), jnp.float32)],
        [row_spec(H), row_spec(H)], grid)

    degp = _make_sc_deg(n_chunks)(dst3d, zeros_hist)
    agg1 = _make_sc_agg(n_chunks)(yg1, src2d, dst3d, zeros_rp)

    deg_spec = pl.BlockSpec((NW, BLK), lambda i: (0, i))
    yg2, z2, rdeg = _tc_call(
        _k2_body, (agg1, degp, z1, gr, br, wt2, b2r),
        [part_spec, deg_spec, row_spec(H), full_spec(gr), full_spec(br),
         full_spec(wt2), full_spec(b2r)],
        [jax.ShapeDtypeStruct((N_PAD, H), jnp.float32),
         jax.ShapeDtypeStruct((N_PAD, H), jnp.float32),
         jax.ShapeDtypeStruct((N_PAD, 16), jnp.float32)],
        [row_spec(H), row_spec(H), row_spec(16)], grid)

    agg2 = _make_sc_agg(n_chunks)(yg2, src2d, dst3d, zeros_rp)

    out = _tc_call(
        _k3_body, (agg2, rdeg, z2),
        [part_spec, row_spec(16), row_spec(H)],
        jax.ShapeDtypeStruct((N_PAD, H), jnp.float32),
        row_spec(H), grid)
    return out[:n]


# speedup vs baseline: 1.0551x; 1.0247x over previous
import functools

import jax
import jax.numpy as jnp
from jax import lax
from jax.experimental import pallas as pl
from jax.experimental.pallas import tpu as pltpu
from jax.experimental.pallas import tpu_sc as plsc

N = 10000
H = 128
LANES = 64
NW = 32
N_PAD = 10240
RPT = N_PAD // 16
DEAD = N_PAD - 8
BLK = 2048


def _k1_body(x_ref, w_ref, b_ref, yg_ref, z_ref):
    xw = jnp.dot(x_ref[...], w_ref[...], preferred_element_type=jnp.float32)
    yg_ref[...] = xw[:, :H]
    z_ref[...] = xw[:, H:] + b_ref[...]


def _k2_body(agg_ref, dp_ref, z1_ref, g_ref, be_ref, w_ref, b2_ref,
             yg2_ref, z2_ref, rdeg_ref):
    p = agg_ref[...]
    dp = dp_ref[...]
    ssum = p[0] + p[1]
    deg = jnp.sum(dp, axis=0)[:, None]
    rdeg = 1.0 / jnp.maximum(deg, 1.0)
    rdeg_ref[...] = jnp.broadcast_to(rdeg, (rdeg.shape[0], 16))
    pre = ssum * rdeg + z1_ref[...]
    mu = jnp.mean(pre, axis=-1, keepdims=True)
    d = pre - mu
    var = jnp.mean(d * d, axis=-1, keepdims=True)
    h = d * lax.rsqrt(var + 1e-5) * g_ref[...] + be_ref[...]
    h = jnp.maximum(h, 0.0)
    hw = jnp.dot(h, w_ref[...], preferred_element_type=jnp.float32)
    yg2_ref[...] = hw[:, :H]
    z2_ref[...] = hw[:, H:] + b2_ref[...]


def _k3_body(agg2_ref, rdeg_ref, z2_ref, out_ref):
    p = agg2_ref[...]
    rdeg = rdeg_ref[...][:, 0:1]
    out_ref[...] = (p[0] + p[1]) * rdeg + z2_ref[...]


def _sc_mesh():
    return plsc.VectorSubcoreMesh(
        core_axis_name="c", subcore_axis_name="s", num_cores=2,
        num_subcores=16)


def _make_sc_agg(n_chunks):
    @functools.partial(
        pl.kernel,
        out_type=jax.ShapeDtypeStruct((2, N_PAD, H), jnp.float32),
        mesh=_sc_mesh(),
        scratch_types=[
            pltpu.VMEM((n_chunks // 2 * LANES,), jnp.int32),
            pltpu.VMEM((n_chunks // 2, LANES), jnp.int32),
            pltpu.VMEM((4, LANES, H), jnp.float32),
            pltpu.VMEM_SHARED((N_PAD, H), jnp.float32),
            pltpu.SemaphoreType.DMA,
            pltpu.SemaphoreType.DMA,
        ],
    )
    def sc_agg(y_hbm, src_hbm, dst_hbm, zeros_hbm, out_hbm,
               src_v, dst_v, rows_v, acc_sh, sem_g, sem_s):
        c = lax.axis_index("c")
        s = lax.axis_index("s")
        wid = s * 2 + c
        pltpu.sync_copy(zeros_hbm, acc_sh.at[pl.ds(s * RPT, RPT)])
        plsc.subcore_barrier()
        bufs = [rows_v.at[b] for b in range(4)]

        def gather(j, b):
            pltpu.async_copy(
                y_hbm.at[src_v.at[pl.ds(j * LANES, LANES)]], bufs[b], sem_g)

        def wait_g(j, b):
            pltpu.make_async_copy(
                y_hbm.at[src_v.at[pl.ds(j * LANES, LANES)]], bufs[b],
                sem_g).wait()

        def scatter(j, b):
            pltpu.async_copy(bufs[b], acc_sh.at[dst_v.at[j]], sem_s, add=True)

        def wait_s(j, b):
            pltpu.make_async_copy(bufs[b], acc_sh.at[dst_v.at[j]], sem_s).wait()

        half = n_chunks // 2
        for p in range(2):
            pltpu.sync_copy(
                src_hbm.at[wid, pl.ds(p * half * LANES, half * LANES)], src_v)
            pltpu.sync_copy(dst_hbm.at[wid, pl.ds(p * half, half)], dst_v)
            gather(0, 0)
            gather(1, 1)
            wait_g(0, 0)
            gather(2, 2)
            scatter(0, 0)
            wait_g(1, 1)
            gather(3, 3)
            scatter(1, 1)
            wait_g(2, 2)
            wait_s(0, 0)
            gather(4, 0)
            scatter(2, 2)
            wait_g(3, 3)
            wait_s(1, 1)
            gather(5, 1)
            scatter(3, 3)

            def quad(k, carry):
                j0 = 4 * k
                wait_g(j0, 0)
                wait_s(j0 - 2, 2)
                gather(j0 + 2, 2)
                scatter(j0, 0)
                wait_g(j0 + 1, 1)
                wait_s(j0 - 1, 3)
                gather(j0 + 3, 3)
                scatter(j0 + 1, 1)
                wait_g(j0 + 2, 2)
                wait_s(j0, 0)
                gather(j0 + 4, 0)
                scatter(j0 + 2, 2)
                wait_g(j0 + 3, 3)
                wait_s(j0 + 1, 1)
                gather(j0 + 5, 1)
                scatter(j0 + 3, 3)
                return carry

            lax.fori_loop(1, half // 4 - 1, quad, 0)
            j0 = half - 4
            wait_g(j0, 0)
            wait_s(j0 - 2, 2)
            gather(j0 + 2, 2)
            scatter(j0, 0)
            wait_g(j0 + 1, 1)
            wait_s(j0 - 1, 3)
            gather(j0 + 3, 3)
            scatter(j0 + 1, 1)
            wait_g(j0 + 2, 2)
            wait_s(j0, 0)
            scatter(j0 + 2, 2)
            wait_g(j0 + 3, 3)
            wait_s(j0 + 1, 1)
            scatter(j0 + 3, 3)
            wait_s(j0 + 2, 2)
            wait_s(j0 + 3, 3)
        plsc.subcore_barrier()
        pltpu.sync_copy(acc_sh.at[pl.ds(s * RPT, RPT)],
                        out_hbm.at[c, pl.ds(s * RPT, RPT)])

    return sc_agg


_HALF_N = N_PAD // 2


def _make_sc_deg(n_chunks):
    @functools.partial(
        pl.kernel,
        out_type=jax.ShapeDtypeStruct((NW, N_PAD), jnp.float32),
        mesh=_sc_mesh(),
        compiler_params=pltpu.CompilerParams(needs_layout_passes=False),
        scratch_types=[
            pltpu.VMEM((n_chunks, LANES), jnp.int32),
            pltpu.VMEM((16 * _HALF_N + 16,), jnp.float32),
            pltpu.VMEM((N_PAD,), jnp.float32),
        ],
    )
    def sc_deg(dst_hbm, zeros_hbm, out_hbm, dst_v, hist_v, deg_v):
        c = lax.axis_index("c")
        s = lax.axis_index("s")
        wid = s * 2 + c
        pltpu.sync_copy(dst_hbm.at[wid], dst_v)
        lane = lax.iota(jnp.int32, 16)
        lane_off = lane * _HALF_N
        trash = 16 * _HALF_N + lane
        ones = jnp.ones((16,), jnp.float32)

        for half in range(2):
            pltpu.sync_copy(zeros_hbm, hist_v.at[pl.ds(0, 16 * _HALF_N)])
            base = half * _HALF_N

            def count(j, carry):
                for k in range(LANES // 16):
                    d = dst_v[j, pl.ds(k * 16, 16)] - base
                    m = (d >= 0) & (d < _HALF_N)
                    idx = jnp.where(m, lane_off + d, trash)
                    cur = plsc.load_gather(hist_v, [idx])
                    plsc.store_scatter(hist_v, [idx], cur + ones)
                return carry

            lax.fori_loop(0, n_chunks, count, 0)

            def lanesum(g, carry):
                acc = hist_v[pl.ds(g * 16, 16)]
                for l in range(1, 16):
                    acc = acc + hist_v[pl.ds(l * _HALF_N + g * 16, 16)]
                deg_v[pl.ds(base + g * 16, 16)] = acc
                return carry

            lax.fori_loop(0, _HALF_N // 16, lanesum, 0)

        pltpu.sync_copy(deg_v, out_hbm.at[wid])

    return sc_deg


def _tc_call(body, in_arrays, in_specs, out_shapes, out_specs, grid):
    return pl.pallas_call(
        body, grid=grid, in_specs=in_specs,
        out_specs=out_specs, out_shape=out_shapes,
    )(*in_arrays)


def kernel(x, edge_index, W1_l, b1_l, W1_r, gamma, beta, W2_l, b2_l, W2_r):
    n, in_dim = x.shape
    e = edge_index.shape[1]
    n_chunks = -(-e // (NW * LANES))
    n_chunks = -(-n_chunks // 8) * 8
    e_pad = NW * n_chunks * LANES

    x = jnp.pad(x, ((0, N_PAD - n), (0, 0)))
    ei = edge_index.astype(jnp.int32)
    pad = e_pad - e
    pad_src = jnp.arange(pad, dtype=jnp.int32) % N
    src2d = jnp.concatenate(
        [ei[0], pad_src]).reshape(NW, n_chunks * LANES)
    pad_dst = N + jnp.arange(pad, dtype=jnp.int32) % (N_PAD - N)
    dst3d = jnp.concatenate(
        [ei[1], pad_dst]).reshape(NW, n_chunks, LANES)
    zeros_rp = jnp.zeros((RPT, H), jnp.float32)
    zeros_hist = jnp.zeros((16 * _HALF_N,), jnp.float32)
    wt1 = jnp.concatenate([W1_l, W1_r], axis=0).T
    wt2 = jnp.concatenate([W2_l, W2_r], axis=0).T
    b1r = b1_l.reshape(1, H)
    b2r = b2_l.reshape(1, H)
    gr = gamma.reshape(1, H)
    br = beta.reshape(1, H)

    grid = (N_PAD // BLK,)
    row_spec = lambda w: pl.BlockSpec((BLK, w), lambda i: (i, 0))
    full_spec = lambda a: pl.BlockSpec(a.shape, lambda i: (0, 0))
    part_spec = pl.BlockSpec((2, BLK, H), lambda i: (0, i, 0))

    yg1, z1 = _tc_call(
        _k1_body, (x, wt1, b1r),
        [row_spec(in_dim), full_spec(wt1), full_spec(b1r)],
        [jax.ShapeDtypeStruct((N_PAD, H), jnp.float32),
         jax.ShapeDtypeStruct((N_PAD, H), jnp.float32)],
        [row_spec(H), row_spec(H)], grid)

    degp = _make_sc_deg(n_chunks)(dst3d, zeros_hist)
    agg1 = _make_sc_agg(n_chunks)(yg1, src2d, dst3d, zeros_rp)

    deg_spec = pl.BlockSpec((NW, BLK), lambda i: (0, i))
    yg2, z2, rdeg = _tc_call(
        _k2_body, (agg1, degp, z1, gr, br, wt2, b2r),
        [part_spec, deg_spec, row_spec(H), full_spec(gr), full_spec(br),
         full_spec(wt2), full_spec(b2r)],
        [jax.ShapeDtypeStruct((N_PAD, H), jnp.float32),
         jax.ShapeDtypeStruct((N_PAD, H), jnp.float32),
         jax.ShapeDtypeStruct((N_PAD, 16), jnp.float32)],
        [row_spec(H), row_spec(H), row_spec(16)], grid)

    agg2 = _make_sc_agg(n_chunks)(yg2, src2d, dst3d, zeros_rp)

    out = _tc_call(
        _k3_body, (agg2, rdeg, z2),
        [part_spec, row_spec(16), row_spec(H)],
        jax.ShapeDtypeStruct((N_PAD, H), jnp.float32),
        row_spec(H), grid)
    return out[:n]


# speedup vs baseline: 1.0659x; 1.0102x over previous
import functools

import jax
import jax.numpy as jnp
from jax import lax
from jax.experimental import pallas as pl
from jax.experimental.pallas import tpu as pltpu
from jax.experimental.pallas import tpu_sc as plsc

N = 10000
H = 128
LANES = 64
NW = 32
N_PAD = 10240
RPT = N_PAD // 16
DEAD = N_PAD - 8
BLK = 5120


def _k1_body(x_ref, w_ref, b_ref, yg_ref, z_ref):
    xw = jnp.dot(x_ref[...], w_ref[...], preferred_element_type=jnp.float32)
    yg_ref[...] = xw[:, :H]
    z_ref[...] = xw[:, H:] + b_ref[...]


def _k2_body(agg_ref, dp_ref, z1_ref, g_ref, be_ref, w_ref, b2_ref,
             yg2_ref, z2_ref, rdeg_ref):
    p = agg_ref[...]
    dp = dp_ref[...]
    ssum = p[0] + p[1]
    deg = jnp.sum(dp, axis=0)[:, None]
    rdeg = 1.0 / jnp.maximum(deg, 1.0)
    rdeg_ref[...] = jnp.broadcast_to(rdeg, (rdeg.shape[0], 16))
    pre = ssum * rdeg + z1_ref[...]
    mu = jnp.mean(pre, axis=-1, keepdims=True)
    d = pre - mu
    var = jnp.mean(d * d, axis=-1, keepdims=True)
    h = d * lax.rsqrt(var + 1e-5) * g_ref[...] + be_ref[...]
    h = jnp.maximum(h, 0.0)
    hw = jnp.dot(h, w_ref[...], preferred_element_type=jnp.float32)
    yg2_ref[...] = hw[:, :H]
    z2_ref[...] = hw[:, H:] + b2_ref[...]


def _k3_body(agg2_ref, rdeg_ref, z2_ref, out_ref):
    p = agg2_ref[...]
    rdeg = rdeg_ref[...][:, 0:1]
    out_ref[...] = (p[0] + p[1]) * rdeg + z2_ref[...]


def _sc_mesh():
    return plsc.VectorSubcoreMesh(
        core_axis_name="c", subcore_axis_name="s", num_cores=2,
        num_subcores=16)


def _make_sc_agg(n_chunks):
    @functools.partial(
        pl.kernel,
        out_type=jax.ShapeDtypeStruct((2, N_PAD, H), jnp.float32),
        mesh=_sc_mesh(),
        scratch_types=[
            pltpu.VMEM((n_chunks // 2 * LANES,), jnp.int32),
            pltpu.VMEM((n_chunks // 2, LANES), jnp.int32),
            pltpu.VMEM((4, LANES, H), jnp.float32),
            pltpu.VMEM_SHARED((N_PAD, H), jnp.float32),
            pltpu.SemaphoreType.DMA,
            pltpu.SemaphoreType.DMA,
        ],
    )
    def sc_agg(y_hbm, src_hbm, dst_hbm, zeros_hbm, out_hbm,
               src_v, dst_v, rows_v, acc_sh, sem_g, sem_s):
        c = lax.axis_index("c")
        s = lax.axis_index("s")
        wid = s * 2 + c
        pltpu.sync_copy(zeros_hbm, acc_sh.at[pl.ds(s * RPT, RPT)])
        plsc.subcore_barrier()
        bufs = [rows_v.at[b] for b in range(4)]

        def gather(j, b):
            pltpu.async_copy(
                y_hbm.at[src_v.at[pl.ds(j * LANES, LANES)]], bufs[b], sem_g)

        def wait_g(j, b):
            pltpu.make_async_copy(
                y_hbm.at[src_v.at[pl.ds(j * LANES, LANES)]], bufs[b],
                sem_g).wait()

        def scatter(j, b):
            pltpu.async_copy(bufs[b], acc_sh.at[dst_v.at[j]], sem_s, add=True)

        def wait_s(j, b):
            pltpu.make_async_copy(bufs[b], acc_sh.at[dst_v.at[j]], sem_s).wait()

        half = n_chunks // 2
        for p in range(2):
            pltpu.sync_copy(
                src_hbm.at[wid, pl.ds(p * half * LANES, half * LANES)], src_v)
            pltpu.sync_copy(dst_hbm.at[wid, pl.ds(p * half, half)], dst_v)
            gather(0, 0)
            gather(1, 1)
            wait_g(0, 0)
            gather(2, 2)
            scatter(0, 0)
            wait_g(1, 1)
            gather(3, 3)
            scatter(1, 1)
            wait_g(2, 2)
            wait_s(0, 0)
            gather(4, 0)
            scatter(2, 2)
            wait_g(3, 3)
            wait_s(1, 1)
            gather(5, 1)
            scatter(3, 3)

            def quad(k, carry):
                j0 = 4 * k
                wait_g(j0, 0)
                wait_s(j0 - 2, 2)
                gather(j0 + 2, 2)
                scatter(j0, 0)
                wait_g(j0 + 1, 1)
                wait_s(j0 - 1, 3)
                gather(j0 + 3, 3)
                scatter(j0 + 1, 1)
                wait_g(j0 + 2, 2)
                wait_s(j0, 0)
                gather(j0 + 4, 0)
                scatter(j0 + 2, 2)
                wait_g(j0 + 3, 3)
                wait_s(j0 + 1, 1)
                gather(j0 + 5, 1)
                scatter(j0 + 3, 3)
                return carry

            lax.fori_loop(1, half // 4 - 1, quad, 0)
            j0 = half - 4
            wait_g(j0, 0)
            wait_s(j0 - 2, 2)
            gather(j0 + 2, 2)
            scatter(j0, 0)
            wait_g(j0 + 1, 1)
            wait_s(j0 - 1, 3)
            gather(j0 + 3, 3)
            scatter(j0 + 1, 1)
            wait_g(j0 + 2, 2)
            wait_s(j0, 0)
            scatter(j0 + 2, 2)
            wait_g(j0 + 3, 3)
            wait_s(j0 + 1, 1)
            scatter(j0 + 3, 3)
            wait_s(j0 + 2, 2)
            wait_s(j0 + 3, 3)
        plsc.subcore_barrier()
        pltpu.sync_copy(acc_sh.at[pl.ds(s * RPT, RPT)],
                        out_hbm.at[c, pl.ds(s * RPT, RPT)])

    return sc_agg


_HALF_N = N_PAD // 2


def _make_sc_deg(n_chunks):
    @functools.partial(
        pl.kernel,
        out_type=jax.ShapeDtypeStruct((NW, N_PAD), jnp.float32),
        mesh=_sc_mesh(),
        compiler_params=pltpu.CompilerParams(needs_layout_passes=False),
        scratch_types=[
            pltpu.VMEM((n_chunks, LANES), jnp.int32),
            pltpu.VMEM((16 * _HALF_N + 16,), jnp.float32),
            pltpu.VMEM((N_PAD,), jnp.float32),
        ],
    )
    def sc_deg(dst_hbm, zeros_hbm, out_hbm, dst_v, hist_v, deg_v):
        c = lax.axis_index("c")
        s = lax.axis_index("s")
        wid = s * 2 + c
        pltpu.sync_copy(dst_hbm.at[wid], dst_v)
        lane = lax.iota(jnp.int32, 16)
        lane_off = lane * _HALF_N
        trash = 16 * _HALF_N + lane
        ones = jnp.ones((16,), jnp.float32)

        for half in range(2):
            pltpu.sync_copy(zeros_hbm, hist_v.at[pl.ds(0, 16 * _HALF_N)])
            base = half * _HALF_N

            def count(j, carry):
                for k in range(LANES // 16):
                    d = dst_v[j, pl.ds(k * 16, 16)] - base
                    m = (d >= 0) & (d < _HALF_N)
                    idx = jnp.where(m, lane_off + d, trash)
                    cur = plsc.load_gather(hist_v, [idx])
                    plsc.store_scatter(hist_v, [idx], cur + ones)
                return carry

            lax.fori_loop(0, n_chunks, count, 0)

            def lanesum(g, carry):
                acc = hist_v[pl.ds(g * 16, 16)]
                for l in range(1, 16):
                    acc = acc + hist_v[pl.ds(l * _HALF_N + g * 16, 16)]
                deg_v[pl.ds(base + g * 16, 16)] = acc
                return carry

            lax.fori_loop(0, _HALF_N // 16, lanesum, 0)

        pltpu.sync_copy(deg_v, out_hbm.at[wid])

    return sc_deg


def _tc_call(body, in_arrays, in_specs, out_shapes, out_specs, grid):
    return pl.pallas_call(
        body, grid=grid, in_specs=in_specs,
        out_specs=out_specs, out_shape=out_shapes,
    )(*in_arrays)


def kernel(x, edge_index, W1_l, b1_l, W1_r, gamma, beta, W2_l, b2_l, W2_r):
    n, in_dim = x.shape
    e = edge_index.shape[1]
    n_chunks = -(-e // (NW * LANES))
    n_chunks = -(-n_chunks // 8) * 8
    e_pad = NW * n_chunks * LANES

    x = jnp.pad(x, ((0, N_PAD - n), (0, 0)))
    ei = edge_index.astype(jnp.int32)
    pad = e_pad - e
    pad_src = jnp.arange(pad, dtype=jnp.int32) % N
    src2d = jnp.concatenate(
        [ei[0], pad_src]).reshape(NW, n_chunks * LANES)
    pad_dst = N + jnp.arange(pad, dtype=jnp.int32) % (N_PAD - N)
    dst3d = jnp.concatenate(
        [ei[1], pad_dst]).reshape(NW, n_chunks, LANES)
    zeros_rp = jnp.zeros((RPT, H), jnp.float32)
    zeros_hist = jnp.zeros((16 * _HALF_N,), jnp.float32)
    wt1 = jnp.concatenate([W1_l, W1_r], axis=0).T
    wt2 = jnp.concatenate([W2_l, W2_r], axis=0).T
    b1r = b1_l.reshape(1, H)
    b2r = b2_l.reshape(1, H)
    gr = gamma.reshape(1, H)
    br = beta.reshape(1, H)

    grid = (N_PAD // BLK,)
    row_spec = lambda w: pl.BlockSpec((BLK, w), lambda i: (i, 0))
    full_spec = lambda a: pl.BlockSpec(a.shape, lambda i: (0, 0))
    part_spec = pl.BlockSpec((2, BLK, H), lambda i: (0, i, 0))

    yg1, z1 = _tc_call(
        _k1_body, (x, wt1, b1r),
        [row_spec(in_dim), full_spec(wt1), full_spec(b1r)],
        [jax.ShapeDtypeStruct((N_PAD, H), jnp.float32),
         jax.ShapeDtypeStruct((N_PAD, H), jnp.float32)],
        [row_spec(H), row_spec(H)], grid)

    degp = _make_sc_deg(n_chunks)(dst3d, zeros_hist)
    agg1 = _make_sc_agg(n_chunks)(yg1, src2d, dst3d, zeros_rp)

    deg_spec = pl.BlockSpec((NW, BLK), lambda i: (0, i))
    yg2, z2, rdeg = _tc_call(
        _k2_body, (agg1, degp, z1, gr, br, wt2, b2r),
        [part_spec, deg_spec, row_spec(H), full_spec(gr), full_spec(br),
         full_spec(wt2), full_spec(b2r)],
        [jax.ShapeDtypeStruct((N_PAD, H), jnp.float32),
         jax.ShapeDtypeStruct((N_PAD, H), jnp.float32),
         jax.ShapeDtypeStruct((N_PAD, 16), jnp.float32)],
        [row_spec(H), row_spec(H), row_spec(16)], grid)

    agg2 = _make_sc_agg(n_chunks)(yg2, src2d, dst3d, zeros_rp)

    out = _tc_call(
        _k3_body, (agg2, rdeg, z2),
        [part_spec, row_spec(16), row_spec(H)],
        jax.ShapeDtypeStruct((N_PAD, H), jnp.float32),
        row_spec(H), grid)
    return out[:n]
